# baseline (speedup 1.0000x reference)
import jax
import jax.numpy as jnp
from jax import lax
from jax.experimental import pallas as pl
from jax.experimental.pallas import tpu as pltpu
from jax.experimental.pallas import tpu_sc as plsc

_K = 128
_BE = 2048
_CCH = 8448

_N_U, _N_I = 100000, 50000

_SC_PARAMS = pltpu.CompilerParams(needs_layout_passes=False)


def _make_segsum(nchunks, n_blocks, cch=_CCH, be=_BE, kk=_K, ones_mode=False):
  ndp = nchunks * cch
  e_pt = n_blocks * be
  nrounds = nchunks // 2
  jpb = be // 16
  rk = kk // 16
  rows_pt = (cch + 16) // 16
  zp = rows_pt + (-rows_pt) % 8
  assert cch % 128 == 0 and nchunks % 2 == 0

  mesh = plsc.VectorSubcoreMesh(core_axis_name="c", subcore_axis_name="s")
  scratch = [pltpu.VMEM_SHARED((cch + 32, 128), jnp.float32)]
  if not ones_mode:
    scratch += [pltpu.VMEM((be,), jnp.int32)] * 2
  scratch += [pltpu.VMEM((be,), jnp.int32)] * 2
  if not ones_mode:
    scratch += [pltpu.VMEM((3 * kk,), jnp.int32)]
  scratch += [pltpu.VMEM((3, kk), jnp.int32)]
  if ones_mode:
    scratch += [pltpu.VMEM((kk, 128), jnp.float32)]
  else:
    scratch += [pltpu.VMEM((kk, 128), jnp.float32) for _ in range(3)]
  scratch += [pltpu.SemaphoreType.DMA] * (5 if ones_mode else 8)

  def body(*args):
    if ones_mode:
      (dstx, ones_in, zeros_in, sums, acc, dstv0, dstv1, dbuf2, onesb,
       st0, st1, s0, s1, s2) = args
      srcv0 = srcv1 = None
      ssems = [s0, s1, s2]
      gbufs = [onesb] * 3
    else:
      (feats, srcx, dstx, zeros_in, sums, acc, srcv0, srcv1, dstv0, dstv1,
       sbuf_all, dbuf2, g0, g1, g2, st0, st1,
       gs0, gs1, gs2, s0, s1, s2) = args
      gbufs = [g0, g1, g2]
      gsems = [gs0, gs1, gs2]
      ssems = [s0, s1, s2]
    stsems = [st0, st1]

    cidx = lax.axis_index("c")
    sidx = lax.axis_index("s")
    trashv = jnp.full((16,), cch, jnp.int32) + sidx
    zrow0 = sidx * rows_pt

    if ones_mode:
      pltpu.sync_copy(ones_in, onesb)
    ziv = jnp.zeros((16,), jnp.int32)
    if not ones_mode:
      for r in range(3 * rk):
        sbuf_all[pl.ds(r * 16, 16)] = ziv
    for x in range(3):
      for r in range(rk):
        dbuf2[x, pl.ds(r * 16, 16)] = trashv

    def reset_row(x):
      for r in range(rk):
        dbuf2[x, pl.ds(r * 16, 16)] = trashv

    def fire(x, k):
      y = (x + 2) % 3
      z = (x + 1) % 3
      if ones_mode:

        @pl.when(k >= 2)
        def _():
          pltpu.make_async_copy(onesb, acc.at[dbuf2.at[z]], ssems[z]).wait()
          reset_row(z)

        pltpu.make_async_copy(onesb, acc.at[dbuf2.at[x]],
                              ssems[x]).start(add=True)
      else:

        @pl.when(k >= 1)
        def _():
          pltpu.make_async_copy(feats.at[sbuf_all.at[pl.ds(y * kk, kk)]],
                                gbufs[y], gsems[y]).wait()
          pltpu.make_async_copy(gbufs[y], acc.at[dbuf2.at[y]],
                                ssems[y]).start(add=True)

        @pl.when(k >= 2)
        def _():
          pltpu.make_async_copy(gbufs[z], acc.at[dbuf2.at[z]],
                                ssems[z]).wait()
          reset_row(z)

        pltpu.make_async_copy(feats.at[sbuf_all.at[pl.ds(x * kk, kk)]],
                              gbufs[x], gsems[x]).start()

    pairs = n_blocks // 2
    tail = n_blocks % 2

    for rnd in range(nrounds):
      base = (2 * rnd + cidx) * cch
      ebase0 = sidx * e_pt

      def stage(iblk, srv, dsv, sem):
        if not ones_mode:
          pltpu.make_async_copy(srcx.at[pl.ds(ebase0 + iblk * be, be)],
                                srv, sem).start()
        pltpu.make_async_copy(dstx.at[pl.ds(ebase0 + iblk * be, be)],
                              dsv, sem).start()

      def stage_wait(srv, dsv, sem):
        if not ones_mode:
          pltpu.make_async_copy(srcx.at[pl.ds(ebase0, be)], srv, sem).wait()
        pltpu.make_async_copy(dstx.at[pl.ds(ebase0, be)], dsv, sem).wait()

      def vb_loop(srv, dsv, carry):
        def step(j, cnt, fc, d, csum, m):
          need = cnt + 16 > kk

          @pl.when(need)
          def _():
            for x in range(3):

              @pl.when(fc % 3 == x)
              def _(x=x):
                fire(x, fc)

          cnt = jnp.where(need, jnp.int32(0), cnt)
          fc = jnp.where(need, fc + 1, fc)
          pos = cnt + csum - 1
          seti = ziv + fc % 3
          if not ones_mode:
            sv = srv[pl.ds(j * 16, 16)]
            plsc.store_scatter(sbuf_all, [pos + (fc % 3) * kk], sv, mask=m)
          plsc.store_scatter(dbuf2, [seti, pos], d - base, mask=m)
          return cnt + csum[15], fc

        def vb(t, carry2):
          cnt, fc = carry2
          j0 = t * 2
          d0 = dsv[pl.ds(j0 * 16, 16)]
          d1 = dsv[pl.ds(j0 * 16 + 16, 16)]
          dl0 = d0 - base
          dl1 = d1 - base
          m0 = plsc.bitcast(dl0, jnp.uint32) < jnp.uint32(cch)
          m1 = plsc.bitcast(dl1, jnp.uint32) < jnp.uint32(cch)
          csum0 = plsc.cumsum(m0.astype(jnp.int32))
          csum1 = plsc.cumsum(m1.astype(jnp.int32))
          cnt, fc = step(j0, cnt, fc, d0, csum0, m0)
          cnt, fc = step(j0 + 1, cnt, fc, d1, csum1, m1)
          return cnt, fc

        return lax.fori_loop(0, jpb // 2, vb, carry)

      pltpu.sync_copy(zeros_in.at[pl.ds(0, zp)],
                      acc.at[pl.ds(zrow0, zp)])
      stage(0, srcv0, dstv0, stsems[0])
      plsc.subcore_barrier()

      def pair_body(b2, carry):
        stage_wait(srcv0, dstv0, stsems[0])
        stage(2 * b2 + 1, srcv1, dstv1, stsems[1])
        carry = vb_loop(srcv0, dstv0, carry)
        stage_wait(srcv1, dstv1, stsems[1])
        stage(jnp.minimum(2 * b2 + 2, n_blocks - 1), srcv0, dstv0,
              stsems[0])
        carry = vb_loop(srcv1, dstv1, carry)
        return carry

      carry = lax.fori_loop(0, pairs, pair_body,
                            (jnp.int32(0), jnp.int32(0)))
      stage_wait(srcv0, dstv0, stsems[0])
      if tail:
        carry = vb_loop(srcv0, dstv0, carry)
      cnt, fc = carry

      for x in range(3):

        @pl.when(fc % 3 == x)
        def _(x=x):
          fire(x, fc)
          y = (x + 2) % 3
          if not ones_mode:
            pltpu.make_async_copy(feats.at[sbuf_all.at[pl.ds(x * kk, kk)]],
                                  gbufs[x], gsems[x]).wait()
            pltpu.make_async_copy(gbufs[x], acc.at[dbuf2.at[x]],
                                  ssems[x]).start(add=True)

          @pl.when(fc >= 1)
          def _():
            pltpu.make_async_copy(gbufs[y], acc.at[dbuf2.at[y]],
                                  ssems[y]).wait()

          pltpu.make_async_copy(gbufs[x], acc.at[dbuf2.at[x]],
                                ssems[x]).wait()
          reset_row(x)
          reset_row(y)

      plsc.subcore_barrier()

      fpt = cch // 16
      frow = sidx * fpt
      pltpu.sync_copy(acc.at[pl.ds(frow, fpt)],
                      sums.at[pl.ds(base + frow, fpt)])
      plsc.subcore_barrier()

  return pl.kernel(
      body, out_type=jax.ShapeDtypeStruct((ndp, 128), jnp.float32),
      mesh=mesh, scratch_types=tuple(scratch), compiler_params=_SC_PARAMS)


_BM = 1000


def _full(shape):
  return pl.BlockSpec(shape, lambda i: tuple(0 for _ in shape))


def _blk(shape):
  return pl.BlockSpec(shape, lambda i: (i,) + tuple(0 for _ in shape[1:]))


def _dot(a, b):
  return jnp.dot(a, b, preferred_element_type=jnp.float32)


def _inv(c_ref):
  return 1.0 / jnp.maximum(c_ref[...][:, 0:1], 1.0)


def _k1i_body(s_ref, c_ref, x_ref, wl, b, wr, wc, p_ref, q_ref):
  h = jnp.maximum(
      _dot(s_ref[...] * _inv(c_ref), wl[...]) + b[...] +
      _dot(x_ref[...], wr[...]), 0.0)
  hp = _dot(h, wc[...])
  p_ref[...] = hp[:, :128]
  q_ref[...] = hp[:, 128:]


def _k1u_body(s1_ref, c1_ref, s2_ref, c2_ref, x_ref,
              a1, a2, wx, b0, wc, o1, o2, o3, o4):
  h = jnp.maximum(
      _dot(s1_ref[...] * _inv(c1_ref), a1[...]) +
      _dot(s2_ref[...] * _inv(c2_ref), a2[...]) +
      _dot(x_ref[...], wx[...]) + b0[...], 0.0)
  hp = _dot(h, wc[...])
  o1[...] = hp[:, 0:128]
  o2[...] = hp[:, 128:256]
  o3[...] = hp[:, 256:384]
  o4[...] = hp[:, 384:512]


def _k2i_body(s_ref, c_ref, q_ref, b2, pw, pb, o_ref):
  o = jnp.maximum(s_ref[...] * _inv(c_ref) + b2[...] + q_ref[...], 0.0)
  o_ref[...] = _dot(o, pw[...]) + pb[...]


def _k2u_body(sv_ref, c1_ref, sf_ref, c2_ref,
              qv_ref, qf_ref, bc, pw, pb, o_ref):
  o = jnp.maximum(
      0.5 * (sv_ref[...] * _inv(c1_ref) + sf_ref[...] * _inv(c2_ref) +
             qv_ref[...] + qf_ref[...]) + bc[...], 0.0)
  o_ref[...] = _dot(o, pw[...]) + pb[...]


def _f32(n, d):
  return jax.ShapeDtypeStruct((n, d), jnp.float32)


def _pad_edges(ei, e_pad, n_dst):
  e = ei.shape[1]
  src = jnp.concatenate(
      [ei[0].astype(jnp.int32), jnp.zeros((e_pad - e,), jnp.int32)])
  dst = jnp.concatenate(
      [ei[1].astype(jnp.int32), jnp.full((e_pad - e,), n_dst, jnp.int32)])
  return src, dst


def kernel(x_user, x_item, ei_rates, ei_rev, ei_fol,
           W1l_r, b1l_r, W1r_r, W2l_r, b2l_r, W2r_r,
           W1l_v, b1l_v, W1r_v, W2l_v, b2l_v, W2r_v,
           W1l_f, b1l_f, W1r_f, W2l_f, b2l_f, W2r_f,
           pW_u, pb_u, pW_i, pb_i):
  nch_u = -(-_N_U // _CCH) + (-(-_N_U // _CCH) % 2)
  nch_i = -(-_N_I // _CCH) + (-(-_N_I // _CCH) % 2)
  eb_r = 16 * 8 * _BE
  eb_f = 16 * 4 * _BE

  src_r, dst_r = _pad_edges(ei_rates, eb_r, _N_I)
  src_v, dst_v = _pad_edges(ei_rev, eb_r, _N_U)
  src_f, dst_f = _pad_edges(ei_fol, eb_f, _N_U)

  zeros128 = jnp.zeros((832, 128), jnp.float32)
  ones_blk = jnp.ones((_K, 128), jnp.float32)

  seg_i = _make_segsum(nch_i, 8)
  seg_uv = _make_segsum(nch_u, 8)
  seg_uf = _make_segsum(nch_u, 4)

  cch2 = 12800
  nc2_u = -(-_N_U // cch2) + (-(-_N_U // cch2) % 2)
  nc2_i = -(-_N_I // cch2) + (-(-_N_I // cch2) % 2)
  cnt_r = _make_segsum(nc2_i, 8, cch=cch2, ones_mode=True)(
      dst_r, ones_blk, zeros128)
  cnt_v = _make_segsum(nc2_u, 8, cch=cch2, ones_mode=True)(
      dst_v, ones_blk, zeros128)
  cnt_f = _make_segsum(nc2_u, 4, cch=cch2, ones_mode=True)(
      dst_f, ones_blk, zeros128)

  s1_r = seg_i(x_user, src_r, dst_r, zeros128)
  s1_v = seg_uv(x_item, src_v, dst_v, zeros128)
  s1_f = seg_uf(x_user, src_f, dst_f, zeros128)

  wcat_i = jnp.concatenate([W2l_v, W2r_r], axis=1)
  p_vi, q_it = pl.pallas_call(
      _k1i_body,
      grid=(_N_I // _BM,),
      in_specs=[_blk((_BM, 128)), _blk((_BM, 128)), _blk((_BM, 128)),
                _full((128, 256)), _full((1, 256)), _full((128, 256)),
                _full((256, 256))],
      out_specs=[_blk((_BM, 128)), _blk((_BM, 128))],
      out_shape=[_f32(_N_I, 128), _f32(_N_I, 128)],
  )(s1_r[:_N_I], cnt_r[:_N_I], x_item, W1l_r,
    b1l_r.reshape(1, -1), W1r_r, wcat_i)

  wcat_u = jnp.concatenate([W2l_r, W2l_f, W2r_v, W2r_f], axis=1)
  wx = 0.5 * (W1r_v + W1r_f)
  b0 = (0.5 * (b1l_v + b1l_f)).reshape(1, -1)
  p_ru, p_fu, q_uv, q_uf = pl.pallas_call(
      _k1u_body,
      grid=(_N_U // _BM,),
      in_specs=[_blk((_BM, 128)), _blk((_BM, 128)),
                _blk((_BM, 128)), _blk((_BM, 128)),
                _blk((_BM, 128)),
                _full((128, 256)), _full((128, 256)), _full((128, 256)),
                _full((1, 256)), _full((256, 512))],
      out_specs=[_blk((_BM, 128))] * 4,
      out_shape=[_f32(_N_U, 128)] * 4,
  )(s1_v[:_N_U], cnt_v[:_N_U], s1_f[:_N_U], cnt_f[:_N_U], x_user,
    0.5 * W1l_v, 0.5 * W1l_f, wx, b0, wcat_u)

  s2_r = seg_i(p_ru, src_r, dst_r, zeros128)
  s2_v = seg_uv(p_vi, src_v, dst_v, zeros128)
  s2_f = seg_uf(p_fu, src_f, dst_f, zeros128)

  out_item = pl.pallas_call(
      _k2i_body,
      grid=(_N_I // _BM,),
      in_specs=[_blk((_BM, 128)), _blk((_BM, 128)), _blk((_BM, 128)),
                _full((1, 128)), _full((128, 128)), _full((1, 128))],
      out_specs=_blk((_BM, 128)),
      out_shape=_f32(_N_I, 128),
  )(s2_r[:_N_I], cnt_r[:_N_I], q_it,
    b2l_r.reshape(1, -1), pW_i, pb_i.reshape(1, -1))

  bc = (0.5 * (b2l_v + b2l_f)).reshape(1, -1)
  out_user = pl.pallas_call(
      _k2u_body,
      grid=(_N_U // _BM,),
      in_specs=[_blk((_BM, 128)), _blk((_BM, 128)),
                _blk((_BM, 128)), _blk((_BM, 128)),
                _blk((_BM, 128)), _blk((_BM, 128)),
                _full((1, 128)), _full((128, 128)), _full((1, 128))],
      out_specs=_blk((_BM, 128)),
      out_shape=_f32(_N_U, 128),
  )(s2_v[:_N_U], cnt_v[:_N_U], s2_f[:_N_U], cnt_f[:_N_U], q_uv, q_uf, bc,
    pW_u, pb_u.reshape(1, -1))

  return (out_user, out_item)


# speedup vs baseline: 1.1408x; 1.1408x over previous
import jax
import jax.numpy as jnp
from jax import lax
from jax.experimental import pallas as pl
from jax.experimental.pallas import tpu as pltpu
from jax.experimental.pallas import tpu_sc as plsc

_K = 128
_BE = 1024
_CCH = 8448

_N_U, _N_I = 100000, 50000

_SC_PARAMS = pltpu.CompilerParams(needs_layout_passes=False)


def _make_segsum(nchunks, n_blocks, cch=_CCH, be=_BE, kk=_K, ones_mode=False):
  ndp = nchunks * cch
  e_pt = n_blocks * be
  nrounds = nchunks // 2
  jpb = be // 16
  rk = kk // 16
  rows_pt = (cch + 16) // 16
  zp = rows_pt + (-rows_pt) % 8
  assert cch % 128 == 0 and nchunks % 2 == 0

  mesh = plsc.VectorSubcoreMesh(core_axis_name="c", subcore_axis_name="s")
  scratch = [pltpu.VMEM_SHARED((cch + 32, 128), jnp.float32)]
  if not ones_mode:
    scratch += [pltpu.VMEM((be,), jnp.int32)] * 2
  scratch += [pltpu.VMEM((be,), jnp.int32)] * 2
  if not ones_mode:
    scratch += [pltpu.VMEM((3 * kk,), jnp.int32)]
  scratch += [pltpu.VMEM((3, kk), jnp.int32)]
  if ones_mode:
    scratch += [pltpu.VMEM((kk, 128), jnp.float32)]
  else:
    scratch += [pltpu.VMEM((kk, 128), jnp.float32) for _ in range(3)]
  scratch += [pltpu.SemaphoreType.DMA] * (5 if ones_mode else 8)

  def body(*args):
    if ones_mode:
      (dstx, ones_in, zeros_in, sums, acc, dstv0, dstv1, dbuf2, onesb,
       st0, st1, s0, s1, s2) = args
      srcv0 = srcv1 = None
      ssems = [s0, s1, s2]
      gbufs = [onesb] * 3
    else:
      (feats, srcx, dstx, zeros_in, sums, acc, srcv0, srcv1, dstv0, dstv1,
       sbuf_all, dbuf2, g0, g1, g2, st0, st1,
       gs0, gs1, gs2, s0, s1, s2) = args
      gbufs = [g0, g1, g2]
      gsems = [gs0, gs1, gs2]
      ssems = [s0, s1, s2]
    stsems = [st0, st1]

    cidx = lax.axis_index("c")
    sidx = lax.axis_index("s")
    trashv = jnp.full((16,), cch, jnp.int32) + sidx
    zrow0 = sidx * rows_pt

    if ones_mode:
      pltpu.sync_copy(ones_in, onesb)
    ziv = jnp.zeros((16,), jnp.int32)
    if not ones_mode:
      for r in range(3 * rk):
        sbuf_all[pl.ds(r * 16, 16)] = ziv
    for x in range(3):
      for r in range(rk):
        dbuf2[x, pl.ds(r * 16, 16)] = trashv

    def reset_row(x):
      for r in range(rk):
        dbuf2[x, pl.ds(r * 16, 16)] = trashv

    def fire(x, k):
      y = (x + 2) % 3
      z = (x + 1) % 3
      if ones_mode:

        @pl.when(k >= 2)
        def _():
          pltpu.make_async_copy(onesb, acc.at[dbuf2.at[z]], ssems[z]).wait()
          reset_row(z)

        pltpu.make_async_copy(onesb, acc.at[dbuf2.at[x]],
                              ssems[x]).start(add=True)
      else:

        @pl.when(k >= 1)
        def _():
          pltpu.make_async_copy(feats.at[sbuf_all.at[pl.ds(y * kk, kk)]],
                                gbufs[y], gsems[y]).wait()
          pltpu.make_async_copy(gbufs[y], acc.at[dbuf2.at[y]],
                                ssems[y]).start(add=True)

        @pl.when(k >= 2)
        def _():
          pltpu.make_async_copy(gbufs[z], acc.at[dbuf2.at[z]],
                                ssems[z]).wait()
          reset_row(z)

        pltpu.make_async_copy(feats.at[sbuf_all.at[pl.ds(x * kk, kk)]],
                              gbufs[x], gsems[x]).start()

    pairs = n_blocks // 2
    tail = n_blocks % 2

    for rnd in range(nrounds):
      base = (2 * rnd + cidx) * cch
      ebase0 = sidx * e_pt

      def stage(iblk, srv, dsv, sem):
        if not ones_mode:
          pltpu.make_async_copy(srcx.at[pl.ds(ebase0 + iblk * be, be)],
                                srv, sem).start()
        pltpu.make_async_copy(dstx.at[pl.ds(ebase0 + iblk * be, be)],
                              dsv, sem).start()

      def stage_wait(srv, dsv, sem):
        if not ones_mode:
          pltpu.make_async_copy(srcx.at[pl.ds(ebase0, be)], srv, sem).wait()
        pltpu.make_async_copy(dstx.at[pl.ds(ebase0, be)], dsv, sem).wait()

      def vb_loop(srv, dsv, carry):
        def vb(j, carry2):
          cnt, fc = carry2
          d = dsv[pl.ds(j * 16, 16)]
          dl = d - base
          m = plsc.bitcast(dl, jnp.uint32) < jnp.uint32(cch)
          csum = plsc.cumsum(m.astype(jnp.int32))
          need = cnt + 16 > kk

          @pl.when(need)
          def _():
            for x in range(3):

              @pl.when(fc % 3 == x)
              def _(x=x):
                fire(x, fc)

          cnt = jnp.where(need, jnp.int32(0), cnt)
          fc = jnp.where(need, fc + 1, fc)
          pos = cnt + csum - 1
          seti = ziv + fc % 3
          if not ones_mode:
            sv = srv[pl.ds(j * 16, 16)]
            plsc.store_scatter(sbuf_all, [pos + (fc % 3) * kk], sv, mask=m)
          plsc.store_scatter(dbuf2, [seti, pos], dl, mask=m)
          return cnt + csum[15], fc

        return lax.fori_loop(0, jpb, vb, carry)

      pltpu.sync_copy(zeros_in.at[pl.ds(0, zp)],
                      acc.at[pl.ds(zrow0, zp)])
      stage(0, srcv0, dstv0, stsems[0])
      plsc.subcore_barrier()

      def pair_body(b2, carry):
        stage_wait(srcv0, dstv0, stsems[0])
        stage(2 * b2 + 1, srcv1, dstv1, stsems[1])
        carry = vb_loop(srcv0, dstv0, carry)
        stage_wait(srcv1, dstv1, stsems[1])
        stage(jnp.minimum(2 * b2 + 2, n_blocks - 1), srcv0, dstv0,
              stsems[0])
        carry = vb_loop(srcv1, dstv1, carry)
        return carry

      carry = lax.fori_loop(0, pairs, pair_body,
                            (jnp.int32(0), jnp.int32(0)))
      stage_wait(srcv0, dstv0, stsems[0])
      if tail:
        carry = vb_loop(srcv0, dstv0, carry)
      cnt, fc = carry

      for x in range(3):

        @pl.when(fc % 3 == x)
        def _(x=x):
          fire(x, fc)
          y = (x + 2) % 3
          if not ones_mode:
            pltpu.make_async_copy(feats.at[sbuf_all.at[pl.ds(x * kk, kk)]],
                                  gbufs[x], gsems[x]).wait()
            pltpu.make_async_copy(gbufs[x], acc.at[dbuf2.at[x]],
                                  ssems[x]).start(add=True)

          @pl.when(fc >= 1)
          def _():
            pltpu.make_async_copy(gbufs[y], acc.at[dbuf2.at[y]],
                                  ssems[y]).wait()

          pltpu.make_async_copy(gbufs[x], acc.at[dbuf2.at[x]],
                                ssems[x]).wait()
          reset_row(x)
          reset_row(y)

      plsc.subcore_barrier()

      fpt = cch // 16
      frow = sidx * fpt
      pltpu.sync_copy(acc.at[pl.ds(frow, fpt)],
                      sums.at[pl.ds(base + frow, fpt)])
      plsc.subcore_barrier()

  return pl.kernel(
      body, out_type=jax.ShapeDtypeStruct((ndp, 128), jnp.float32),
      mesh=mesh, scratch_types=tuple(scratch), compiler_params=_SC_PARAMS)


_BM = 1000


def _full(shape):
  return pl.BlockSpec(shape, lambda i: tuple(0 for _ in shape))


def _blk(shape):
  return pl.BlockSpec(shape, lambda i: (i,) + tuple(0 for _ in shape[1:]))


def _dot(a, b):
  return jnp.dot(a, b, preferred_element_type=jnp.float32)


def _inv(c_ref):
  return 1.0 / jnp.maximum(c_ref[...][:, 0:1], 1.0)


def _k1i_body(s_ref, c_ref, x_ref, wl, b, wr, wc, p_ref, q_ref):
  h = jnp.maximum(
      _dot(s_ref[...] * _inv(c_ref), wl[...]) + b[...] +
      _dot(x_ref[...], wr[...]), 0.0)
  hp = _dot(h, wc[...])
  p_ref[...] = hp[:, :128]
  q_ref[...] = hp[:, 128:]


def _k1u_body(s1_ref, c1_ref, s2_ref, c2_ref, x_ref,
              a1, a2, wx, b0, wc, o1, o2, o3, o4):
  h = jnp.maximum(
      _dot(s1_ref[...] * _inv(c1_ref), a1[...]) +
      _dot(s2_ref[...] * _inv(c2_ref), a2[...]) +
      _dot(x_ref[...], wx[...]) + b0[...], 0.0)
  hp = _dot(h, wc[...])
  o1[...] = hp[:, 0:128]
  o2[...] = hp[:, 128:256]
  o3[...] = hp[:, 256:384]
  o4[...] = hp[:, 384:512]


def _k2i_body(s_ref, c_ref, q_ref, b2, pw, pb, o_ref):
  o = jnp.maximum(s_ref[...] * _inv(c_ref) + b2[...] + q_ref[...], 0.0)
  o_ref[...] = _dot(o, pw[...]) + pb[...]


def _k2u_body(sv_ref, c1_ref, sf_ref, c2_ref,
              qv_ref, qf_ref, bc, pw, pb, o_ref):
  o = jnp.maximum(
      0.5 * (sv_ref[...] * _inv(c1_ref) + sf_ref[...] * _inv(c2_ref) +
             qv_ref[...] + qf_ref[...]) + bc[...], 0.0)
  o_ref[...] = _dot(o, pw[...]) + pb[...]


def _f32(n, d):
  return jax.ShapeDtypeStruct((n, d), jnp.float32)


def _pad_edges(ei, e_pad, n_dst):
  e = ei.shape[1]
  src = jnp.concatenate(
      [ei[0].astype(jnp.int32), jnp.zeros((e_pad - e,), jnp.int32)])
  dst = jnp.concatenate(
      [ei[1].astype(jnp.int32), jnp.full((e_pad - e,), n_dst, jnp.int32)])
  return src, dst


def kernel(x_user, x_item, ei_rates, ei_rev, ei_fol,
           W1l_r, b1l_r, W1r_r, W2l_r, b2l_r, W2r_r,
           W1l_v, b1l_v, W1r_v, W2l_v, b2l_v, W2r_v,
           W1l_f, b1l_f, W1r_f, W2l_f, b2l_f, W2r_f,
           pW_u, pb_u, pW_i, pb_i):
  nch_u = -(-_N_U // _CCH) + (-(-_N_U // _CCH) % 2)
  nch_i = -(-_N_I // _CCH) + (-(-_N_I // _CCH) % 2)
  eb_r = 16 * 16 * _BE
  eb_f = 16 * 7 * _BE

  src_r, dst_r = _pad_edges(ei_rates, eb_r, _N_I)
  src_v, dst_v = _pad_edges(ei_rev, eb_r, _N_U)
  src_f, dst_f = _pad_edges(ei_fol, eb_f, _N_U)

  zeros128 = jnp.zeros((832, 128), jnp.float32)
  ones_blk = jnp.ones((_K, 128), jnp.float32)

  seg_i = _make_segsum(nch_i, 16)
  seg_uv = _make_segsum(nch_u, 16)
  seg_uf = _make_segsum(nch_u, 7)

  cch2 = 12800
  nc2_u = -(-_N_U // cch2) + (-(-_N_U // cch2) % 2)
  nc2_i = -(-_N_I // cch2) + (-(-_N_I // cch2) % 2)
  cnt_r = _make_segsum(nc2_i, 16, cch=cch2, ones_mode=True)(
      dst_r, ones_blk, zeros128)
  cnt_v = _make_segsum(nc2_u, 16, cch=cch2, ones_mode=True)(
      dst_v, ones_blk, zeros128)
  cnt_f = _make_segsum(nc2_u, 7, cch=cch2, ones_mode=True)(
      dst_f, ones_blk, zeros128)

  s1_r = seg_i(x_user, src_r, dst_r, zeros128)
  s1_v = seg_uv(x_item, src_v, dst_v, zeros128)
  s1_f = seg_uf(x_user, src_f, dst_f, zeros128)

  wcat_i = jnp.concatenate([W2l_v, W2r_r], axis=1)
  p_vi, q_it = pl.pallas_call(
      _k1i_body,
      grid=(_N_I // _BM,),
      in_specs=[_blk((_BM, 128)), _blk((_BM, 128)), _blk((_BM, 128)),
                _full((128, 256)), _full((1, 256)), _full((128, 256)),
                _full((256, 256))],
      out_specs=[_blk((_BM, 128)), _blk((_BM, 128))],
      out_shape=[_f32(_N_I, 128), _f32(_N_I, 128)],
  )(s1_r[:_N_I], cnt_r[:_N_I], x_item, W1l_r,
    b1l_r.reshape(1, -1), W1r_r, wcat_i)

  wcat_u = jnp.concatenate([W2l_r, W2l_f, W2r_v, W2r_f], axis=1)
  wx = 0.5 * (W1r_v + W1r_f)
  b0 = (0.5 * (b1l_v + b1l_f)).reshape(1, -1)
  p_ru, p_fu, q_uv, q_uf = pl.pallas_call(
      _k1u_body,
      grid=(_N_U // _BM,),
      in_specs=[_blk((_BM, 128)), _blk((_BM, 128)),
                _blk((_BM, 128)), _blk((_BM, 128)),
                _blk((_BM, 128)),
                _full((128, 256)), _full((128, 256)), _full((128, 256)),
                _full((1, 256)), _full((256, 512))],
      out_specs=[_blk((_BM, 128))] * 4,
      out_shape=[_f32(_N_U, 128)] * 4,
  )(s1_v[:_N_U], cnt_v[:_N_U], s1_f[:_N_U], cnt_f[:_N_U], x_user,
    0.5 * W1l_v, 0.5 * W1l_f, wx, b0, wcat_u)

  s2_r = seg_i(p_ru, src_r, dst_r, zeros128)
  s2_v = seg_uv(p_vi, src_v, dst_v, zeros128)
  s2_f = seg_uf(p_fu, src_f, dst_f, zeros128)

  out_item = pl.pallas_call(
      _k2i_body,
      grid=(_N_I // _BM,),
      in_specs=[_blk((_BM, 128)), _blk((_BM, 128)), _blk((_BM, 128)),
                _full((1, 128)), _full((128, 128)), _full((1, 128))],
      out_specs=_blk((_BM, 128)),
      out_shape=_f32(_N_I, 128),
  )(s2_r[:_N_I], cnt_r[:_N_I], q_it,
    b2l_r.reshape(1, -1), pW_i, pb_i.reshape(1, -1))

  bc = (0.5 * (b2l_v + b2l_f)).reshape(1, -1)
  out_user = pl.pallas_call(
      _k2u_body,
      grid=(_N_U // _BM,),
      in_specs=[_blk((_BM, 128)), _blk((_BM, 128)),
                _blk((_BM, 128)), _blk((_BM, 128)),
                _blk((_BM, 128)), _blk((_BM, 128)),
                _full((1, 128)), _full((128, 128)), _full((1, 128))],
      out_specs=_blk((_BM, 128)),
      out_shape=_f32(_N_U, 128),
  )(s2_v[:_N_U], cnt_v[:_N_U], s2_f[:_N_U], cnt_f[:_N_U], q_uv, q_uf, bc,
    pW_u, pb_u.reshape(1, -1))

  return (out_user, out_item)
